# baseline (speedup 1.0000x reference)
import functools

import jax
import jax.numpy as jnp
from jax import lax
from jax.experimental import pallas as pl
from jax.experimental.pallas import tpu as pltpu
from jax.experimental.pallas import tpu_sc as plsc

N_TOK = 9216
N_CODES = 8192
D = 256

BZ = 1024
BE = 1024
CH = 128
NCH = BE // CH
NJ = N_CODES // BE


def _process(mm_get, jj, zn, en_ref, rmin_ref, rarg_ref):
    accv = None
    for c in range(NCH):
        en_c = en_ref[:, pl.ds(jj * BE + c * CH, CH)]
        dv = (zn + en_c) - mm_get(c)
        da = jnp.full((BZ, CH), 0.0, jnp.float32) + (jj * BE + c * CH).astype(jnp.float32)
        if accv is None:
            accv, acca = dv, da
        else:
            better = dv < accv
            accv = jnp.where(better, dv, accv)
            acca = jnp.where(better, da, acca)
    rv = rmin_ref[...]
    ra = rarg_ref[...]
    better = accv < rv
    rmin_ref[...] = jnp.where(better, accv, rv)
    rarg_ref[...] = jnp.where(better, acca, ra)


def _argmin_body(z2_ref, et_ref, idx_ref,
                 mm_ref, rmin_ref, rarg_ref, zn_ref, en_ref):
    i = pl.program_id(0)
    j = pl.program_id(1)
    z2 = z2_ref[...]
    et = et_ref[...]

    @pl.when(j == 0)
    def _():
        zn_ref[...] = 0.25 * jnp.sum(z2 * z2, axis=1, keepdims=True)
        rmin_ref[...] = jnp.full((BZ, CH), 3e38, jnp.float32)
        rarg_ref[...] = jnp.zeros((BZ, CH), jnp.float32)

    @pl.when(i == 0)
    def _():
        en_ref[:, pl.ds(j * BE, BE)] = jnp.sum(et * et, axis=0, keepdims=True)

    mm2 = lax.dot_general(z2, et, (((1,), (0,)), ((), ())),
                          preferred_element_type=jnp.float32)
    mm_ref[j % 2] = mm2

    pen = jnp.where(j == 0, 3e38, 0.0).astype(jnp.float32)
    zn = zn_ref[...] + pen
    jj = jnp.maximum(j - 1, 0)
    prev = (j + 1) % 2
    _process(lambda c: mm_ref[prev, :, c * CH:(c + 1) * CH],
             jj, zn, en_ref, rmin_ref, rarg_ref)

    @pl.when(j == NJ - 1)
    def _():
        zn0 = zn_ref[...]
        _process(lambda c: mm2[:, c * CH:(c + 1) * CH],
                 j, zn0, en_ref, rmin_ref, rarg_ref)
        rv = rmin_ref[...]
        gm = jnp.min(rv, axis=1, keepdims=True)
        lanef = lax.broadcasted_iota(jnp.int32, (BZ, CH), 1).astype(jnp.float32)
        cand = jnp.where(rv == gm, rarg_ref[...] + lanef, 3e38)
        idx_ref[...] = jnp.min(cand, axis=1, keepdims=True).astype(jnp.int32)


def _argmin_call(z2, emb_t):
    grid = (N_TOK // BZ, NJ)
    return pl.pallas_call(
        _argmin_body,
        grid=grid,
        in_specs=[
            pl.BlockSpec((BZ, D), lambda i, j: (i, 0)),
            pl.BlockSpec((D, BE), lambda i, j: (0, j)),
        ],
        out_specs=pl.BlockSpec((BZ, 1), lambda i, j: (i, 0)),
        out_shape=jax.ShapeDtypeStruct((N_TOK, 1), jnp.int32),
        scratch_shapes=[
            pltpu.VMEM((2, BZ, BE), jnp.float32),
            pltpu.VMEM((BZ, CH), jnp.float32),
            pltpu.VMEM((BZ, CH), jnp.float32),
            pltpu.VMEM((BZ, 1), jnp.float32),
            pltpu.VMEM((1, N_CODES), jnp.float32),
        ],
        compiler_params=pltpu.CompilerParams(
            dimension_semantics=("parallel", "arbitrary"),
        ),
    )(z2, emb_t)


_NW = 32
_BPW = N_TOK // _NW


def _gather_call(embedding, idx):
    mesh = plsc.VectorSubcoreMesh(core_axis_name="c", subcore_axis_name="s")

    @functools.partial(
        pl.kernel,
        mesh=mesh,
        out_type=jax.ShapeDtypeStruct((N_TOK, D), jnp.float32),
        scratch_types=[
            pltpu.VMEM((_BPW,), jnp.int32),
            pltpu.VMEM((_BPW, D), jnp.float32),
            pltpu.SemaphoreType.DMA,
        ],
    )
    def k(table_hbm, idx_hbm, out_hbm, idx_v, rows_v, sem):
        wid = lax.axis_index("s") * 2 + lax.axis_index("c")
        base = wid * _BPW
        pltpu.sync_copy(idx_hbm.at[pl.ds(base, _BPW)], idx_v)
        pltpu.async_copy(table_hbm.at[idx_v], rows_v, sem).wait()
        pltpu.sync_copy(rows_v, out_hbm.at[pl.ds(base, _BPW)])

    return k(embedding, idx)


def kernel(z, embedding):
    z2 = z + z
    emb_t = embedding.T
    idx = _argmin_call(z2, emb_t).reshape(N_TOK)
    z_q = _gather_call(embedding, idx)
    return z_q, idx


# speedup vs baseline: 1.4740x; 1.4740x over previous
import functools

import jax
import jax.numpy as jnp
from jax import lax
from jax.experimental import pallas as pl
from jax.experimental.pallas import tpu as pltpu
from jax.experimental.pallas import tpu_sc as plsc

N_TOK = 9216
N_CODES = 8192
D = 256
KA = 384

BZ = 512
SB = 1024
NS = N_CODES // SB
CH = 128
NCH = SB // CH


def _merge(av, aa, bv, ba):
    better = bv < av
    return jnp.where(better, bv, av), jnp.where(better, ba, aa)


def _argmin_body(z_ref, et_ref, idx_ref, eaug_ref):
    i = pl.program_id(0)

    @pl.when(i == 0)
    def _():
        et = et_ref[...]
        eaug_ref[0:D, :] = et
        eaug_ref[D:D + 1, :] = jnp.ones((1, N_CODES), jnp.float32)
        eaug_ref[D + 1:D + 2, :] = jnp.sum(et * et, axis=0, keepdims=True)
        eaug_ref[D + 2:KA, :] = jnp.zeros((KA - D - 2, N_CODES), jnp.float32)

    z = z_ref[...]
    zn = jnp.sum(z * z, axis=1, keepdims=True)
    zaug = jnp.concatenate(
        [-(z + z), zn, jnp.ones((BZ, 1), jnp.float32),
         jnp.zeros((BZ, KA - D - 2), jnp.float32)], axis=1)

    accv = acca = None
    for s in range(NS):
        dmat = lax.dot_general(
            zaug, eaug_ref[:, s * SB:(s + 1) * SB],
            (((1,), (0,)), ((), ())), preferred_element_type=jnp.float32)
        pairs = []
        for c in range(NCH):
            dv = dmat[:, c * CH:(c + 1) * CH]
            da = jnp.full((BZ, CH), float(s * SB + c * CH), jnp.float32)
            pairs.append((dv, da))
        while len(pairs) > 1:
            pairs = [_merge(*pairs[k], *pairs[k + 1])
                     for k in range(0, len(pairs), 2)]
        sv, sa = pairs[0]
        if accv is None:
            accv, acca = sv, sa
        else:
            accv, acca = _merge(accv, acca, sv, sa)

    gm = jnp.min(accv, axis=1, keepdims=True)
    lanef = lax.broadcasted_iota(jnp.int32, (BZ, CH), 1).astype(jnp.float32)
    cand = jnp.where(accv == gm, acca + lanef, 3e38)
    idx_ref[...] = jnp.min(cand, axis=1, keepdims=True).astype(jnp.int32)


def _argmin_call(z, emb_t):
    grid = (N_TOK // BZ,)
    return pl.pallas_call(
        _argmin_body,
        grid=grid,
        in_specs=[
            pl.BlockSpec((BZ, D), lambda i: (i, 0)),
            pl.BlockSpec((D, N_CODES), lambda i: (0, 0)),
        ],
        out_specs=pl.BlockSpec((BZ, 1), lambda i: (i, 0)),
        out_shape=jax.ShapeDtypeStruct((N_TOK, 1), jnp.int32),
        scratch_shapes=[
            pltpu.VMEM((KA, N_CODES), jnp.float32),
        ],
        compiler_params=pltpu.CompilerParams(
            dimension_semantics=("arbitrary",),
        ),
    )(z, emb_t)


_NW = 32
_BPW = N_TOK // _NW


def _gather_call(embedding, idx):
    mesh = plsc.VectorSubcoreMesh(core_axis_name="c", subcore_axis_name="s")

    @functools.partial(
        pl.kernel,
        mesh=mesh,
        out_type=jax.ShapeDtypeStruct((N_TOK, D), jnp.float32),
        scratch_types=[
            pltpu.VMEM((_BPW,), jnp.int32),
            pltpu.VMEM((_BPW, D), jnp.float32),
            pltpu.SemaphoreType.DMA,
        ],
    )
    def k(table_hbm, idx_hbm, out_hbm, idx_v, rows_v, sem):
        wid = lax.axis_index("s") * 2 + lax.axis_index("c")
        base = wid * _BPW
        pltpu.sync_copy(idx_hbm.at[pl.ds(base, _BPW)], idx_v)
        pltpu.async_copy(table_hbm.at[idx_v], rows_v, sem).wait()
        pltpu.sync_copy(rows_v, out_hbm.at[pl.ds(base, _BPW)])

    return k(embedding, idx)


def kernel(z, embedding):
    emb_t = embedding.T
    idx = _argmin_call(z, emb_t).reshape(N_TOK)
    z_q = _gather_call(embedding, idx)
    return z_q, idx


# speedup vs baseline: 1.5368x; 1.0426x over previous
import functools

import jax
import jax.numpy as jnp
from jax import lax
from jax.experimental import pallas as pl
from jax.experimental.pallas import tpu as pltpu
from jax.experimental.pallas import tpu_sc as plsc

N_TOK = 9216
N_CODES = 8192
D = 256
KA = 320

BZ = 1152
SB = 1024
NS = N_CODES // SB
CH = 128
NCH = SB // CH


def _merge(av, aa, bv, ba):
    better = bv < av
    return jnp.where(better, bv, av), jnp.where(better, ba, aa)


def _argmin_body(z_ref, et_ref, idx_ref, eaug_ref):
    i = pl.program_id(0)

    @pl.when(i == 0)
    def _():
        et = et_ref[...]
        eaug_ref[0:D, :] = et
        eaug_ref[D:D + 1, :] = jnp.ones((1, N_CODES), jnp.float32)
        eaug_ref[D + 1:D + 2, :] = jnp.sum(et * et, axis=0, keepdims=True)
        eaug_ref[D + 2:KA, :] = jnp.zeros((KA - D - 2, N_CODES), jnp.float32)

    z = z_ref[...]
    zn = jnp.sum(z * z, axis=1, keepdims=True)
    zaug = jnp.concatenate(
        [-(z + z), zn, jnp.ones((BZ, 1), jnp.float32),
         jnp.zeros((BZ, KA - D - 2), jnp.float32)], axis=1)

    accv = acca = None
    for s in range(NS):
        dmat = lax.dot_general(
            zaug, eaug_ref[:, s * SB:(s + 1) * SB],
            (((1,), (0,)), ((), ())), preferred_element_type=jnp.float32)
        sv = sa = None
        for c in range(NCH):
            dv = dmat[:, c * CH:(c + 1) * CH]
            da = jnp.full((BZ, CH), float(s * SB + c * CH), jnp.float32)
            if sv is None:
                sv, sa = dv, da
            else:
                sv, sa = _merge(sv, sa, dv, da)
        if accv is None:
            accv, acca = sv, sa
        else:
            accv, acca = _merge(accv, acca, sv, sa)

    gm = jnp.min(accv, axis=1, keepdims=True)
    lanef = lax.broadcasted_iota(jnp.int32, (BZ, CH), 1).astype(jnp.float32)
    cand = jnp.where(accv == gm, acca + lanef, 3e38)
    idx_ref[...] = jnp.min(cand, axis=1, keepdims=True).astype(jnp.int32)


def _argmin_call(z, emb_t):
    grid = (N_TOK // BZ,)
    return pl.pallas_call(
        _argmin_body,
        grid=grid,
        in_specs=[
            pl.BlockSpec((BZ, D), lambda i: (i, 0)),
            pl.BlockSpec((D, N_CODES), lambda i: (0, 0)),
        ],
        out_specs=pl.BlockSpec((BZ, 1), lambda i: (i, 0)),
        out_shape=jax.ShapeDtypeStruct((N_TOK, 1), jnp.int32),
        scratch_shapes=[
            pltpu.VMEM((KA, N_CODES), jnp.float32),
        ],
        compiler_params=pltpu.CompilerParams(
            dimension_semantics=("arbitrary",),
        ),
    )(z, emb_t)


_NW = 32
_BPW = N_TOK // _NW


def _gather_call(embedding, idx):
    mesh = plsc.VectorSubcoreMesh(core_axis_name="c", subcore_axis_name="s")

    @functools.partial(
        pl.kernel,
        mesh=mesh,
        out_type=jax.ShapeDtypeStruct((N_TOK, D), jnp.float32),
        scratch_types=[
            pltpu.VMEM((_BPW,), jnp.int32),
            pltpu.VMEM((_BPW, D), jnp.float32),
            pltpu.SemaphoreType.DMA,
        ],
    )
    def k(table_hbm, idx_hbm, out_hbm, idx_v, rows_v, sem):
        wid = lax.axis_index("s") * 2 + lax.axis_index("c")
        base = wid * _BPW
        pltpu.sync_copy(idx_hbm.at[pl.ds(base, _BPW)], idx_v)
        pltpu.async_copy(table_hbm.at[idx_v], rows_v, sem).wait()
        pltpu.sync_copy(rows_v, out_hbm.at[pl.ds(base, _BPW)])

    return k(embedding, idx)


def kernel(z, embedding):
    emb_t = embedding.T
    idx = _argmin_call(z, emb_t).reshape(N_TOK)
    z_q = _gather_call(embedding, idx)
    return z_q, idx


# speedup vs baseline: 1.8185x; 1.1832x over previous
import functools

import jax
import jax.numpy as jnp
from jax import lax
from jax.experimental import pallas as pl
from jax.experimental.pallas import tpu as pltpu
from jax.experimental.pallas import tpu_sc as plsc

N_TOK = 9216
N_CODES = 8192
D = 256

BZ = 1152
SB = 1024
NS = N_CODES // SB
CH = 128
NCH = SB // CH


def _merge(av, aa, bv, ba):
    better = bv < av
    return jnp.where(better, bv, av), jnp.where(better, ba, aa)


def _argmin_body(z_ref, et_ref, idx_ref, en_ref):
    i = pl.program_id(0)

    @pl.when(i == 0)
    def _():
        et = et_ref[...]
        en_ref[...] = jnp.sum(et * et, axis=0, keepdims=True)

    z = z_ref[...]
    zn = jnp.sum(z * z, axis=1, keepdims=True)
    zm2 = -(z + z)

    accv = acca = None
    for s in range(NS):
        m2 = lax.dot_general(
            zm2, et_ref[:, s * SB:(s + 1) * SB],
            (((1,), (0,)), ((), ())), preferred_element_type=jnp.float32)
        sv = sa = None
        for c in range(NCH):
            col0 = s * SB + c * CH
            dv = (zn + en_ref[:, col0:col0 + CH]) + m2[:, c * CH:(c + 1) * CH]
            da = jnp.full((BZ, CH), float(col0), jnp.float32)
            if sv is None:
                sv, sa = dv, da
            else:
                sv, sa = _merge(sv, sa, dv, da)
        if accv is None:
            accv, acca = sv, sa
        else:
            accv, acca = _merge(accv, acca, sv, sa)

    gm = jnp.min(accv, axis=1, keepdims=True)
    lanef = lax.broadcasted_iota(jnp.int32, (BZ, CH), 1).astype(jnp.float32)
    cand = jnp.where(accv == gm, acca + lanef, 3e38)
    idx_ref[...] = jnp.min(cand, axis=1, keepdims=True).astype(jnp.int32)


def _argmin_call(z, emb_t):
    grid = (N_TOK // BZ,)
    return pl.pallas_call(
        _argmin_body,
        grid=grid,
        in_specs=[
            pl.BlockSpec((BZ, D), lambda i: (i, 0)),
            pl.BlockSpec((D, N_CODES), lambda i: (0, 0)),
        ],
        out_specs=pl.BlockSpec((BZ, 1), lambda i: (i, 0)),
        out_shape=jax.ShapeDtypeStruct((N_TOK, 1), jnp.int32),
        scratch_shapes=[
            pltpu.VMEM((1, N_CODES), jnp.float32),
        ],
        compiler_params=pltpu.CompilerParams(
            dimension_semantics=("arbitrary",),
        ),
    )(z, emb_t)


_NW = 32
_BPW = N_TOK // _NW


def _gather_call(embedding, idx):
    mesh = plsc.VectorSubcoreMesh(core_axis_name="c", subcore_axis_name="s")

    @functools.partial(
        pl.kernel,
        mesh=mesh,
        out_type=jax.ShapeDtypeStruct((N_TOK, D), jnp.float32),
        scratch_types=[
            pltpu.VMEM((_BPW,), jnp.int32),
            pltpu.VMEM((_BPW, D), jnp.float32),
            pltpu.SemaphoreType.DMA,
        ],
    )
    def k(table_hbm, idx_hbm, out_hbm, idx_v, rows_v, sem):
        wid = lax.axis_index("s") * 2 + lax.axis_index("c")
        base = wid * _BPW
        pltpu.sync_copy(idx_hbm.at[pl.ds(base, _BPW)], idx_v)
        pltpu.async_copy(table_hbm.at[idx_v], rows_v, sem).wait()
        pltpu.sync_copy(rows_v, out_hbm.at[pl.ds(base, _BPW)])

    return k(embedding, idx)


def kernel(z, embedding):
    emb_t = embedding.T
    idx = _argmin_call(z, emb_t).reshape(N_TOK)
    z_q = _gather_call(embedding, idx)
    return z_q, idx


# speedup vs baseline: 1.8855x; 1.0369x over previous
import functools

import jax
import jax.numpy as jnp
from jax import lax
from jax.experimental import pallas as pl
from jax.experimental.pallas import tpu as pltpu
from jax.experimental.pallas import tpu_sc as plsc

N_TOK = 9216
N_CODES = 8192
D = 256

BZ = 2304
SB = 1024
NS = N_CODES // SB
CH = 128
NCH = SB // CH


def _merge(av, aa, bv, ba):
    better = bv < av
    return jnp.where(better, bv, av), jnp.where(better, ba, aa)


def _argmin_body(z_ref, et_ref, idx_ref, en_ref):
    i = pl.program_id(0)

    @pl.when(i == 0)
    def _():
        et = et_ref[...]
        en_ref[...] = jnp.sum(et * et, axis=0, keepdims=True)

    z = z_ref[...]
    zn = jnp.sum(z * z, axis=1, keepdims=True)
    zm2 = -(z + z)

    accv = acca = None
    for s in range(NS):
        m2 = lax.dot_general(
            zm2, et_ref[:, s * SB:(s + 1) * SB],
            (((1,), (0,)), ((), ())), preferred_element_type=jnp.float32)
        sv = sa = None
        for c in range(NCH):
            col0 = s * SB + c * CH
            dv = (zn + en_ref[:, col0:col0 + CH]) + m2[:, c * CH:(c + 1) * CH]
            da = jnp.full((BZ, CH), float(col0), jnp.float32)
            if sv is None:
                sv, sa = dv, da
            else:
                sv, sa = _merge(sv, sa, dv, da)
        if accv is None:
            accv, acca = sv, sa
        else:
            accv, acca = _merge(accv, acca, sv, sa)

    gm = jnp.min(accv, axis=1, keepdims=True)
    lanef = lax.broadcasted_iota(jnp.int32, (BZ, CH), 1).astype(jnp.float32)
    cand = jnp.where(accv == gm, acca + lanef, 3e38)
    idx_ref[...] = jnp.min(cand, axis=1, keepdims=True).astype(jnp.int32)


def _argmin_call(z, emb_t):
    grid = (N_TOK // BZ,)
    return pl.pallas_call(
        _argmin_body,
        grid=grid,
        in_specs=[
            pl.BlockSpec((BZ, D), lambda i: (i, 0)),
            pl.BlockSpec((D, N_CODES), lambda i: (0, 0)),
        ],
        out_specs=pl.BlockSpec((BZ, 1), lambda i: (i, 0)),
        out_shape=jax.ShapeDtypeStruct((N_TOK, 1), jnp.int32),
        scratch_shapes=[
            pltpu.VMEM((1, N_CODES), jnp.float32),
        ],
        compiler_params=pltpu.CompilerParams(
            dimension_semantics=("arbitrary",),
        ),
    )(z, emb_t)


_NW = 32
_BPW = N_TOK // _NW


def _gather_call(embedding, idx):
    mesh = plsc.VectorSubcoreMesh(core_axis_name="c", subcore_axis_name="s")

    @functools.partial(
        pl.kernel,
        mesh=mesh,
        out_type=jax.ShapeDtypeStruct((N_TOK, D), jnp.float32),
        scratch_types=[
            pltpu.VMEM((_BPW,), jnp.int32),
            pltpu.VMEM((_BPW, D), jnp.float32),
            pltpu.SemaphoreType.DMA,
        ],
    )
    def k(table_hbm, idx_hbm, out_hbm, idx_v, rows_v, sem):
        wid = lax.axis_index("s") * 2 + lax.axis_index("c")
        base = wid * _BPW
        pltpu.sync_copy(idx_hbm.at[pl.ds(base, _BPW)], idx_v)
        pltpu.async_copy(table_hbm.at[idx_v], rows_v, sem).wait()
        pltpu.sync_copy(rows_v, out_hbm.at[pl.ds(base, _BPW)])

    return k(embedding, idx)


def kernel(z, embedding):
    emb_t = embedding.T
    idx = _argmin_call(z, emb_t).reshape(N_TOK)
    z_q = _gather_call(embedding, idx)
    return z_q, idx
